# baseline (speedup 1.0000x reference)
import functools

import jax
import jax.numpy as jnp
from jax import lax
from jax.experimental import pallas as pl
from jax.experimental.pallas import tpu as pltpu
from jax.experimental.pallas import tpu_sc as plsc

_LANE = 128


def kernel(seq_len, relative_embeddings):
    del seq_len
    two_max_len, embed = relative_embeddings.shape
    s = two_max_len // 2
    g_cols = two_max_len - _LANE

    info = plsc.get_sparse_core_info()
    num_workers = info.num_cores * info.num_subcores
    classes_per_w = _LANE // num_workers
    rows_per_class = s // _LANE

    table_t = relative_embeddings.T

    def build_shifted(tab_ref, out_ref):
        for c in range(_LANE):
            out_ref[c] = tab_ref[:, _LANE - c:two_max_len - c]

    shifted = pl.pallas_call(
        build_shifted,
        out_shape=jax.ShapeDtypeStruct((_LANE, embed, g_cols), jnp.float32),
    )(table_t)

    mesh = plsc.VectorSubcoreMesh(core_axis_name="c", subcore_axis_name="s")

    @functools.partial(
        pl.kernel,
        mesh=mesh,
        out_type=jax.ShapeDtypeStruct((s, embed, s), jnp.float32),
        scratch_types=[
            pltpu.VMEM((1, embed, g_cols), jnp.float32),
            pltpu.VMEM((1, embed, g_cols), jnp.float32),
            pltpu.SemaphoreType.DMA,
            pltpu.SemaphoreType.DMA,
            pltpu.SemaphoreType.DMA,
        ],
    )
    def toeplitz_planes(
        shifted_hbm, out_hbm, table_a, table_b, stage_sem, sem_a, sem_b
    ):
        wid = lax.axis_index("s") * info.num_cores + lax.axis_index("c")
        bufs = (table_a, table_b)
        sems = (sem_a, sem_b)

        def stage(j):
            pltpu.async_copy(
                shifted_hbm.at[pl.ds(classes_per_w * wid + j, 1)],
                bufs[j % 2],
                stage_sem,
            )

        def stage_wait(j):
            pltpu.make_async_copy(
                shifted_hbm.at[pl.ds(0, 1)], bufs[j % 2], stage_sem
            ).wait()

        def planes(j, fire):
            c = classes_per_w * wid + j
            buf = bufs[j % 2]
            sem = sems[j % 2]

            def body(k, carry):
                i = c + _LANE * k
                a = pl.multiple_of(s - _LANE - _LANE * k, _LANE)
                copy = pltpu.make_async_copy(
                    buf.at[:, :, pl.ds(a, s)], out_hbm.at[pl.ds(i, 1)], sem
                )
                if fire:
                    copy.start()
                else:
                    copy.wait()
                return carry

            lax.fori_loop(0, rows_per_class, body, 0)

        stage(0)
        for j in range(classes_per_w):
            stage_wait(j)
            planes(j, fire=True)
            if j >= 1:
                planes(j - 1, fire=False)
            if j + 1 < classes_per_w:
                stage(j + 1)
        planes(classes_per_w - 1, fire=False)

    return toeplitz_planes(shifted).transpose(0, 2, 1)
